# baseline (speedup 1.0000x reference)
import functools

import jax
import jax.numpy as jnp
from jax import lax
from jax.experimental import pallas as pl
from jax.experimental.pallas import tpu as pltpu
from jax.experimental.pallas import tpu_sc as plsc

N = 10000
M = 200
E = 160000
EM = 2000
F = 128
H = 128
R = 4

NS = 16
NC = 2
NP = 10240
EP = 163840
EROWS = EP // 128
NB = 10
BN = 1000

_mesh = plsc.VectorSubcoreMesh(core_axis_name="c", subcore_axis_name="s")
_sc_params = pltpu.CompilerParams(use_tc_tiling_on_sc=False)


def _zero_vmem(buf, rows, width):
    per = width // 16

    def st(i, _):
        j = i // per
        k = (i % per) * 16
        buf[j, pl.ds(k, 16)] = jnp.zeros((16,), jnp.float32)
        return _

    lax.fori_loop(0, rows * per, st, 0)


def _pipelined_gather_scatter(table, A, B, rows0, rows1, acc,
                              semg0, semg1, sems0, sems1, K):
    pltpu.async_copy(table.at[A.at[0]], rows0, semg0)
    pltpu.async_copy(table.at[A.at[1]], rows1, semg1)

    def outer(g, carry):
        i0 = 2 * g
        pltpu.make_async_copy(table.at[A.at[i0]], rows0, semg0).wait()
        pltpu.async_copy(rows0, acc.at[B.at[i0]], sems0, add=True)
        pltpu.make_async_copy(table.at[A.at[i0 + 1]], rows1, semg1).wait()
        pltpu.async_copy(rows1, acc.at[B.at[i0 + 1]], sems1, add=True)
        pltpu.make_async_copy(rows0, acc.at[B.at[i0]], sems0).wait()

        @pl.when(i0 + 2 < K)
        def _f0():
            pltpu.async_copy(table.at[A.at[i0 + 2]], rows0, semg0)

        pltpu.make_async_copy(rows1, acc.at[B.at[i0 + 1]], sems1).wait()

        @pl.when(i0 + 3 < K)
        def _f1():
            pltpu.async_copy(table.at[A.at[i0 + 3]], rows1, semg1)

        return carry

    lax.fori_loop(0, K // 2, outer, 0)


def _sc_pct_body(et2d, src2d, dst2d, table, out, A, B, rows, rows1, zbuf,
                 acc, sem, sem1, sem2, sem3):
    c = lax.axis_index("c")
    s = lax.axis_index("s")
    wid = c * NS + s
    rows_per_tile = EROWS // (NC * NS)
    base = wid * rows_per_tile

    _zero_vmem(zbuf, 16, 128)

    def z1(j, _):
        pltpu.sync_copy(zbuf, acc.at[pl.ds(s * 640 + j * 16, 16)])
        return _
    lax.fori_loop(0, 40, z1, 0)
    plsc.subcore_barrier()

    pltpu.sync_copy(et2d.at[pl.ds(base, rows_per_tile)], A)
    pltpu.sync_copy(src2d.at[pl.ds(base, rows_per_tile)], B)

    def gix(i, _):
        j = i // 8
        k = (i % 8) * 16
        A[j, pl.ds(k, 16)] = A[j, pl.ds(k, 16)] * N + B[j, pl.ds(k, 16)]
        return _
    lax.fori_loop(0, rows_per_tile * 8, gix, 0)
    pltpu.sync_copy(dst2d.at[pl.ds(base, rows_per_tile)], B)

    _pipelined_gather_scatter(table, A, B, rows, rows1, acc, sem, sem1,
                              sem2, sem3, rows_per_tile)
    plsc.subcore_barrier()

    pltpu.sync_copy(acc.at[pl.ds(s * 640, 640)],
                    out.at[c].at[pl.ds(s * 640, 640)])


_sc_pct = functools.partial(
    pl.kernel,
    out_type=jax.ShapeDtypeStruct((NC, NP, 128), jnp.float32),
    mesh=_mesh,
    scratch_types=[
        pltpu.VMEM((40, 128), jnp.int32),
        pltpu.VMEM((40, 128), jnp.int32),
        pltpu.VMEM((128, 128), jnp.float32),
        pltpu.VMEM((128, 128), jnp.float32),
        pltpu.VMEM((16, 128), jnp.float32),
        pltpu.VMEM_SHARED((NP, 128), jnp.float32),
        pltpu.SemaphoreType.DMA,
        pltpu.SemaphoreType.DMA,
        pltpu.SemaphoreType.DMA,
        pltpu.SemaphoreType.DMA,
    ],
    compiler_params=_sc_params,
)(_sc_pct_body)


def _sc_hist_body(et2d, dst2d, hist, A, B, zbuf, obuf, acch, sem):
    c = lax.axis_index("c")
    s = lax.axis_index("s")
    wid = c * NS + s
    rows_per_tile = EROWS // (NC * NS)
    base = wid * rows_per_tile

    _zero_vmem(zbuf, 64, 16)

    def z2(j, _):
        pltpu.sync_copy(zbuf, acch.at[pl.ds(s * 2560 + j * 64, 64)])
        return _
    lax.fori_loop(0, 40, z2, 0)

    def o1(j, _):
        obuf[j, pl.ds(0, 16)] = jnp.ones((16,), jnp.float32)
        return _
    lax.fori_loop(0, 128, o1, 0)
    plsc.subcore_barrier()

    pltpu.sync_copy(et2d.at[pl.ds(base, rows_per_tile)], A)
    pltpu.sync_copy(dst2d.at[pl.ds(base, rows_per_tile)], B)

    def hix(i, _):
        j = i // 8
        k = (i % 8) * 16
        A[j, pl.ds(k, 16)] = A[j, pl.ds(k, 16)] * NP + B[j, pl.ds(k, 16)]
        return _
    lax.fori_loop(0, rows_per_tile * 8, hix, 0)

    def hstep(j, _):
        pltpu.sync_copy(obuf, acch.at[A.at[j]], add=True)
        return _
    lax.fori_loop(0, rows_per_tile, hstep, 0)
    plsc.subcore_barrier()

    pltpu.sync_copy(acch.at[pl.ds(s * 2560, 2560)],
                    hist.at[c].at[pl.ds(s * 2560, 2560)])


_sc_hist = functools.partial(
    pl.kernel,
    out_type=jax.ShapeDtypeStruct((NC, R * NP, 16), jnp.float32),
    mesh=_mesh,
    scratch_types=[
        pltpu.VMEM((40, 128), jnp.int32),
        pltpu.VMEM((40, 128), jnp.int32),
        pltpu.VMEM((64, 16), jnp.float32),
        pltpu.VMEM((128, 16), jnp.float32),
        pltpu.VMEM_SHARED((R * NP, 16), jnp.float32),
        pltpu.SemaphoreType.DMA,
    ],
    compiler_params=_sc_params,
)(_sc_hist_body)


def _sc_ppi_body(et2d, src2d, dst2d, table32, out, A, B, C, rows32, rows32b,
                 zbuf32, acc32, sem, sem1, sem2, sem3):
    c = lax.axis_index("c")
    s = lax.axis_index("s")
    rows_per_tile = EROWS // NS
    base = s * rows_per_tile

    _zero_vmem(zbuf32, 64, 32)
    pltpu.sync_copy(et2d.at[pl.ds(base, rows_per_tile)], A)
    pltpu.sync_copy(src2d.at[pl.ds(base, rows_per_tile)], C)
    pltpu.sync_copy(dst2d.at[pl.ds(base, rows_per_tile)], B)

    def six(i, _):
        j = i // 8
        k = (i % 8) * 16
        B[j, pl.ds(k, 16)] = A[j, pl.ds(k, 16)] * NP + B[j, pl.ds(k, 16)]
        return _
    lax.fori_loop(0, rows_per_tile * 8, six, 0)

    def gix(i, _):
        j = i // 8
        k = (i % 8) * 16
        C[j, pl.ds(k, 16)] = (A[j, pl.ds(k, 16)] * N + C[j, pl.ds(k, 16)]) * 4
        return _
    lax.fori_loop(0, rows_per_tile * 8, gix, 0)

    for q in (0, 1):
        chunk = 2 * c + q

        def z1(j, _):
            pltpu.sync_copy(zbuf32, acc32.at[pl.ds(s * 2560 + j * 64, 64)])
            return _
        lax.fori_loop(0, 40, z1, 0)

        def cix(i, _):
            j = i // 8
            k = (i % 8) * 16
            A[j, pl.ds(k, 16)] = C[j, pl.ds(k, 16)] + chunk
            return _
        lax.fori_loop(0, rows_per_tile * 8, cix, 0)
        plsc.subcore_barrier()

        _pipelined_gather_scatter(table32, A, B, rows32, rows32b, acc32,
                                  sem, sem1, sem2, sem3, rows_per_tile)
        plsc.subcore_barrier()

        pltpu.sync_copy(acc32.at[pl.ds(s * 2560, 2560)],
                        out.at[chunk].at[pl.ds(s * 2560, 2560)])
        plsc.subcore_barrier()


_sc_ppi = functools.partial(
    pl.kernel,
    out_type=jax.ShapeDtypeStruct((4, R * NP, 32), jnp.float32),
    mesh=_mesh,
    scratch_types=[
        pltpu.VMEM((80, 128), jnp.int32),
        pltpu.VMEM((80, 128), jnp.int32),
        pltpu.VMEM((80, 128), jnp.int32),
        pltpu.VMEM((128, 32), jnp.float32),
        pltpu.VMEM((128, 32), jnp.float32),
        pltpu.VMEM((64, 32), jnp.float32),
        pltpu.VMEM_SHARED((R * NP, 32), jnp.float32),
        pltpu.SemaphoreType.DMA,
        pltpu.SemaphoreType.DMA,
        pltpu.SemaphoreType.DMA,
        pltpu.SemaphoreType.DMA,
    ],
    compiler_params=_sc_params,
)(_sc_ppi_body)


def _relmm_body(x_ref, w_ref, o_ref):
    o_ref[...] = jnp.dot(x_ref[...], w_ref[0],
                         preferred_element_type=jnp.float32)[None]


def _relmm(x, W):
    return pl.pallas_call(
        _relmm_body,
        grid=(R, NB),
        in_specs=[
            pl.BlockSpec((BN, 128), lambda r, i: (i, 0)),
            pl.BlockSpec((1, 128, H), lambda r, i: (r, 0, 0)),
        ],
        out_specs=pl.BlockSpec((1, BN, H), lambda r, i: (r, i, 0)),
        out_shape=jax.ShapeDtypeStruct((R, N, H), jnp.float32),
        compiler_params=pltpu.CompilerParams(
            dimension_semantics=("parallel", "parallel")),
    )(x, W)


def _make_mgk(apply_relu):
    tdims = (((0,), (0,)), ((), ()))

    def body(x_ref, tiss_ref, mgx_ref, wmg_ref, wcross_ref, wpool_ref,
             relw_ref, msrc_ref, mdst_ref, met_ref,
             mgout_ref, mgt_ref, pool_acc, cnt_acc):
        i = pl.program_id(0)

        @pl.when(i == 0)
        def _():
            pool_acc[...] = jnp.zeros_like(pool_acc)
            cnt_acc[...] = jnp.zeros_like(cnt_acc)

        tb = tiss_ref[...]
        oh = (tb == lax.broadcasted_iota(jnp.int32, (BN, M), 1)).astype(
            jnp.float32)
        pool_acc[...] += lax.dot_general(
            oh, x_ref[...], tdims, preferred_element_type=jnp.float32)
        cnt_acc[...] += lax.dot_general(
            oh, jnp.ones((BN, 128), jnp.float32), tdims,
            preferred_element_type=jnp.float32)

        @pl.when(i == NB - 1)
        def _():
            mgx = mgx_ref[...]
            t = jnp.dot(mgx, wmg_ref[...], preferred_element_type=jnp.float32)
            msrc = msrc_ref[...]
            ohs = (msrc == lax.broadcasted_iota(jnp.int32, (EM, M), 1)).astype(
                jnp.float32)
            met = met_ref[...]
            ohe = (met == lax.broadcasted_iota(jnp.int32, (EM, R), 1)).astype(
                jnp.float32)
            mm = jnp.dot(ohs, t, preferred_element_type=jnp.float32) * jnp.dot(
                ohe, relw_ref[...], preferred_element_type=jnp.float32)
            mdst = mdst_ref[...]
            ohd = (mdst == lax.broadcasted_iota(jnp.int32, (EM, M), 1)).astype(
                jnp.float32)
            magg = lax.dot_general(ohd, mm, tdims,
                                   preferred_element_type=jnp.float32)
            mdeg = lax.dot_general(ohd, jnp.ones((EM, 128), jnp.float32),
                                   tdims, preferred_element_type=jnp.float32)
            pooled = pool_acc[...] / jnp.maximum(cnt_acc[...], 1.0)
            res = magg / jnp.maximum(mdeg, 1.0) + jnp.dot(
                pooled, wpool_ref[...], preferred_element_type=jnp.float32)
            if apply_relu:
                res = jnp.maximum(res, 0.0)
            mgout_ref[...] = res
            mgt_ref[...] = jnp.dot(mgx, wcross_ref[...],
                                   preferred_element_type=jnp.float32)

    return pl.pallas_call(
        body,
        grid=(NB,),
        in_specs=[
            pl.BlockSpec((BN, 128), lambda i: (i, 0)),
            pl.BlockSpec((BN, 1), lambda i: (i, 0)),
            pl.BlockSpec((M, 128), lambda i: (0, 0)),
            pl.BlockSpec((128, H), lambda i: (0, 0)),
            pl.BlockSpec((128, H), lambda i: (0, 0)),
            pl.BlockSpec((128, H), lambda i: (0, 0)),
            pl.BlockSpec((R, H), lambda i: (0, 0)),
            pl.BlockSpec((EM, 1), lambda i: (0, 0)),
            pl.BlockSpec((EM, 1), lambda i: (0, 0)),
            pl.BlockSpec((EM, 1), lambda i: (0, 0)),
        ],
        out_specs=[
            pl.BlockSpec((M, H), lambda i: (0, 0)),
            pl.BlockSpec((M, H), lambda i: (0, 0)),
        ],
        out_shape=[
            jax.ShapeDtypeStruct((M, H), jnp.float32),
            jax.ShapeDtypeStruct((M, H), jnp.float32),
        ],
        scratch_shapes=[
            pltpu.VMEM((M, 128), jnp.float32),
            pltpu.VMEM((M, 128), jnp.float32),
        ],
    )


def _pct_post_body(agg_ref, hist_ref, tiss_ref, mgt_ref, wq_ref, o_ref):
    h = hist_ref[...]
    deg = jnp.sum(h[..., 0], axis=(0, 1))
    agg = agg_ref[0] + agg_ref[1]
    t = tiss_ref[...]
    oh = (t == lax.broadcasted_iota(jnp.int32, (BN, M), 1)).astype(jnp.float32)
    p = agg / jnp.maximum(deg, 1.0)[:, None] + jnp.dot(
        oh, mgt_ref[...], preferred_element_type=jnp.float32)
    for r in range(R):
        o_ref[r] = jnp.dot(p, wq_ref[r], preferred_element_type=jnp.float32)


def _pct_post(agg, hist, tiss, mgt, Wq):
    return pl.pallas_call(
        _pct_post_body,
        grid=(NB,),
        in_specs=[
            pl.BlockSpec((NC, BN, 128), lambda i: (0, i, 0)),
            pl.BlockSpec((NC, R, BN, 16), lambda i: (0, 0, i, 0)),
            pl.BlockSpec((BN, 1), lambda i: (i, 0)),
            pl.BlockSpec((M, H), lambda i: (0, 0)),
            pl.BlockSpec((R, 128, H), lambda i: (0, 0, 0)),
        ],
        out_specs=pl.BlockSpec((R, BN, H), lambda i: (0, i, 0)),
        out_shape=jax.ShapeDtypeStruct((R, N, H), jnp.float32),
        compiler_params=pltpu.CompilerParams(
            dimension_semantics=("parallel",)),
    )(agg, hist, tiss, mgt, Wq)


def _make_ppi_post_combine(with_next):
    def body(agg_ref, hist_ref, wa_ref, ba_ref, qa_ref, wp_ref,
             *refs):
        if with_next:
            xr_ref, p_ref, aggn_s, wr_s = refs
        else:
            p_ref, aggn_s, wr_s = refs
        i = pl.program_id(0)

        @pl.when(i < NB)
        def _phase1():
            a = agg_ref[...]
            h = hist_ref[...]
            cnt = h[0, :, :, 0] + h[1, :, :, 0]
            full = jnp.concatenate([a[0], a[1], a[2], a[3]], axis=-1)
            aggn = full / jnp.maximum(cnt, 1.0)[:, :, None]
            aggn_s[:, pl.ds(i * BN, BN), :] = aggn
            sco = jnp.tanh(
                jnp.dot(aggn.reshape(R * BN, H), wa_ref[...],
                        preferred_element_type=jnp.float32) + ba_ref[...])
            pv = jnp.sum(sco * qa_ref[...], axis=-1).reshape(R, BN)
            s4 = jnp.sum(pv, axis=1)
            col0 = (lax.broadcasted_iota(jnp.int32, (R, 128), 1) == 0
                    ).astype(jnp.float32)
            contrib = jnp.concatenate(
                [s4[:, None] * col0, jnp.zeros((4, 128), jnp.float32)],
                axis=0)

            @pl.when(i == 0)
            def _():
                wr_s[...] = jnp.zeros_like(wr_s)

            wr_s[...] += contrib

        @pl.when(i >= NB)
        def _phase2():
            j = i - NB
            aggn = aggn_s[:, pl.ds(j * BN, BN), :]
            p = _beta_weighted(aggn, wr_s[...])
            if with_next:
                p = jnp.maximum(p, 0.0)
                p_ref[...] = p
                for r in range(R):
                    xr_ref[r] = jnp.dot(p, wp_ref[r],
                                        preferred_element_type=jnp.float32)
            else:
                p_ref[...] = p

    def blk(i):
        return jnp.where(i < NB, i, i - NB)

    def out_blk(i):
        return jnp.maximum(i - NB, 0)

    out_specs = [pl.BlockSpec((BN, H), lambda i: (out_blk(i), 0))]
    out_shape = [jax.ShapeDtypeStruct((N, H), jnp.float32)]
    if with_next:
        out_specs = [pl.BlockSpec((R, BN, H), lambda i: (0, out_blk(i), 0))
                     ] + out_specs
        out_shape = [jax.ShapeDtypeStruct((R, N, H), jnp.float32)] + out_shape

    def run(agg, hist, Wa, ba, qa, Wp):
        return pl.pallas_call(
            body,
            grid=(2 * NB,),
            in_specs=[
                pl.BlockSpec((4, R, BN, 32), lambda i: (0, 0, blk(i), 0)),
                pl.BlockSpec((NC, R, BN, 16), lambda i: (0, 0, blk(i), 0)),
                pl.BlockSpec((H, 8), lambda i: (0, 0)),
                pl.BlockSpec((1, 8), lambda i: (0, 0)),
                pl.BlockSpec((1, 8), lambda i: (0, 0)),
                pl.BlockSpec((R, 128, H), lambda i: (0, 0, 0)),
            ],
            out_specs=out_specs,
            out_shape=out_shape,
            scratch_shapes=[
                pltpu.VMEM((R, N, H), jnp.float32),
                pltpu.VMEM((8, 128), jnp.float32),
            ],
            compiler_params=pltpu.CompilerParams(
                vmem_limit_bytes=56 * 1024 * 1024),
        )(agg, hist, Wa, ba, qa, Wp)

    return run


_ppi_combine_next = _make_ppi_post_combine(True)
_ppi_combine_last = _make_ppi_post_combine(False)


def _beta_weighted(aggn_blk, wr_blk):
    w = wr_blk[:, 0:1] / float(N)
    rowmask = lax.broadcasted_iota(jnp.int32, (8, 1), 0) < R
    m = jnp.max(jnp.where(rowmask, w, -1e30))
    e = jnp.where(rowmask, jnp.exp(w - m), 0.0)
    beta = e / jnp.sum(e)
    return jnp.sum(aggn_blk * beta[0:R].reshape(R, 1, 1), axis=0)


_mgk_relu = _make_mgk(True)
_mgk_final = _make_mgk(False)


def kernel(ppi_x, metagraph_x, ppi_edgetypes, mg_edgetypes, ppi_edge_index,
           mg_edge_index, tissue_neighbors, relw, Wp1, Wmg1, Wcross1, Wpool1,
           Wq1, Wa1, ba1, qa1, Wp2, Wmg2, Wcross2, Wpool2, Wq2, Wa2, ba2,
           qa2):
    et = ppi_edgetypes.astype(jnp.int32)
    src = ppi_edge_index[0].astype(jnp.int32)
    dst = ppi_edge_index[1].astype(jnp.int32)
    pad = EP - E
    et2d = jnp.concatenate([et, jnp.full((pad,), R - 1, jnp.int32)]
                           ).reshape(EROWS, 128)
    src2d = jnp.concatenate([src, jnp.zeros((pad,), jnp.int32)]
                            ).reshape(EROWS, 128)
    dst2d = jnp.concatenate([dst, jnp.full((pad,), N, jnp.int32)]
                            ).reshape(EROWS, 128)

    tiss = tissue_neighbors.astype(jnp.int32)
    tiss_col = tiss.reshape(N, 1)
    msrc = mg_edge_index[0].astype(jnp.int32).reshape(EM, 1)
    mdst_col = mg_edge_index[1].astype(jnp.int32).reshape(EM, 1)
    met = mg_edgetypes.astype(jnp.int32).reshape(EM, 1)
    ba1r = ba1.reshape(1, 8)
    qa1r = qa1.reshape(1, 8)
    ba2r = ba2.reshape(1, 8)
    qa2r = qa2.reshape(1, 8)

    hist4 = _sc_hist(et2d, dst2d).reshape(NC, R, NP, 16)

    xr1 = _relmm(ppi_x, Wp1)
    agg1 = _sc_pct(et2d, src2d, dst2d, xr1.reshape(R * N, H))
    mg1, mgt1 = _mgk_relu(ppi_x, tiss_col, metagraph_x, Wmg1, Wcross1,
                          Wpool1, relw, msrc, mdst_col, met)
    xr2_1 = _pct_post(agg1, hist4, tiss_col, mgt1, Wq1)
    agg2_1 = _sc_ppi(et2d, src2d, dst2d, xr2_1.reshape(R * N * 4, 32))
    xr1_l2, p1relu = _ppi_combine_next(agg2_1.reshape(4, R, NP, 32), hist4,
                                       Wa1, ba1r, qa1r, Wp2)

    agg1_2 = _sc_pct(et2d, src2d, dst2d, xr1_l2.reshape(R * N, H))
    mg2, mgt2 = _mgk_final(p1relu, tiss_col, mg1, Wmg2, Wcross2, Wpool2,
                           relw, msrc, mdst_col, met)
    xr2_2 = _pct_post(agg1_2, hist4, tiss_col, mgt2, Wq2)
    agg2_2 = _sc_ppi(et2d, src2d, dst2d, xr2_2.reshape(R * N * 4, 32))
    p2, = _ppi_combine_last(agg2_2.reshape(4, R, NP, 32), hist4, Wa2, ba2r,
                            qa2r, Wp2)
    return (p2, mg2)


# speedup vs baseline: 1.0663x; 1.0663x over previous
import functools

import jax
import jax.numpy as jnp
from jax import lax
from jax.experimental import pallas as pl
from jax.experimental.pallas import tpu as pltpu
from jax.experimental.pallas import tpu_sc as plsc

N = 10000
M = 200
E = 160000
EM = 2000
F = 128
H = 128
R = 4

NS = 16
NC = 2
NP = 10240
EP = 163840
EROWS = EP // 128
NB = 10
BN = 1000

_mesh = plsc.VectorSubcoreMesh(core_axis_name="c", subcore_axis_name="s")
_sc_params = pltpu.CompilerParams(use_tc_tiling_on_sc=False)


def _zero_vmem(buf, rows, width):
    per = width // 16

    def st(i, _):
        j = i // per
        k = (i % per) * 16
        buf[j, pl.ds(k, 16)] = jnp.zeros((16,), jnp.float32)
        return _

    lax.fori_loop(0, rows * per, st, 0)


def _pipelined_gather_scatter(table, A, B, rows0, rows1, acc,
                              semg0, semg1, sems0, sems1, K):
    pltpu.async_copy(table.at[A.at[0]], rows0, semg0)
    pltpu.async_copy(table.at[A.at[1]], rows1, semg1)

    def outer(g, carry):
        i0 = 2 * g
        pltpu.make_async_copy(table.at[A.at[i0]], rows0, semg0).wait()
        pltpu.async_copy(rows0, acc.at[B.at[i0]], sems0, add=True)
        pltpu.make_async_copy(table.at[A.at[i0 + 1]], rows1, semg1).wait()
        pltpu.async_copy(rows1, acc.at[B.at[i0 + 1]], sems1, add=True)
        pltpu.make_async_copy(rows0, acc.at[B.at[i0]], sems0).wait()

        @pl.when(i0 + 2 < K)
        def _f0():
            pltpu.async_copy(table.at[A.at[i0 + 2]], rows0, semg0)

        pltpu.make_async_copy(rows1, acc.at[B.at[i0 + 1]], sems1).wait()

        @pl.when(i0 + 3 < K)
        def _f1():
            pltpu.async_copy(table.at[A.at[i0 + 3]], rows1, semg1)

        return carry

    lax.fori_loop(0, K // 2, outer, 0)


def _sc_pct_body(et2d, src2d, dst2d, table, out, A, B, rows, rows1, zbuf,
                 acc, sem, sem1, sem2, sem3):
    c = lax.axis_index("c")
    s = lax.axis_index("s")
    wid = c * NS + s
    rows_per_tile = EROWS // (NC * NS)
    base = wid * rows_per_tile

    _zero_vmem(zbuf, 16, 128)

    def z1(j, _):
        pltpu.async_copy(zbuf, acc.at[pl.ds(s * 640 + j * 16, 16)], sem)
        return _
    lax.fori_loop(0, 40, z1, 0)

    def z1w(j, _):
        pltpu.make_async_copy(zbuf, acc.at[pl.ds(s * 640, 16)], sem).wait()
        return _
    lax.fori_loop(0, 40, z1w, 0)
    plsc.subcore_barrier()

    pltpu.sync_copy(et2d.at[pl.ds(base, rows_per_tile)], A)
    pltpu.sync_copy(src2d.at[pl.ds(base, rows_per_tile)], B)

    def gix(i, _):
        j = i // 8
        k = (i % 8) * 16
        A[j, pl.ds(k, 16)] = A[j, pl.ds(k, 16)] * N + B[j, pl.ds(k, 16)]
        return _
    lax.fori_loop(0, rows_per_tile * 8, gix, 0)
    pltpu.sync_copy(dst2d.at[pl.ds(base, rows_per_tile)], B)

    _pipelined_gather_scatter(table, A, B, rows, rows1, acc, sem, sem1,
                              sem2, sem3, rows_per_tile)
    plsc.subcore_barrier()

    pltpu.sync_copy(acc.at[pl.ds(s * 640, 640)],
                    out.at[c].at[pl.ds(s * 640, 640)])


_sc_pct = functools.partial(
    pl.kernel,
    out_type=jax.ShapeDtypeStruct((NC, NP, 128), jnp.float32),
    mesh=_mesh,
    scratch_types=[
        pltpu.VMEM((40, 128), jnp.int32),
        pltpu.VMEM((40, 128), jnp.int32),
        pltpu.VMEM((128, 128), jnp.float32),
        pltpu.VMEM((128, 128), jnp.float32),
        pltpu.VMEM((16, 128), jnp.float32),
        pltpu.VMEM_SHARED((NP, 128), jnp.float32),
        pltpu.SemaphoreType.DMA,
        pltpu.SemaphoreType.DMA,
        pltpu.SemaphoreType.DMA,
        pltpu.SemaphoreType.DMA,
    ],
    compiler_params=_sc_params,
)(_sc_pct_body)


def _sc_hist_body(et2d, dst2d, hist, A, B, zbuf, obuf, acch, sem):
    c = lax.axis_index("c")
    s = lax.axis_index("s")
    wid = c * NS + s
    rows_per_tile = EROWS // (NC * NS)
    base = wid * rows_per_tile

    _zero_vmem(zbuf, 64, 16)

    def z2(j, _):
        pltpu.async_copy(zbuf, acch.at[pl.ds(s * 2560 + j * 64, 64)], sem)
        return _
    lax.fori_loop(0, 40, z2, 0)

    def o1(j, _):
        obuf[j, pl.ds(0, 16)] = jnp.ones((16,), jnp.float32)
        return _
    lax.fori_loop(0, 128, o1, 0)

    def z2w(j, _):
        pltpu.make_async_copy(zbuf, acch.at[pl.ds(s * 2560, 64)], sem).wait()
        return _
    lax.fori_loop(0, 40, z2w, 0)
    plsc.subcore_barrier()

    pltpu.sync_copy(et2d.at[pl.ds(base, rows_per_tile)], A)
    pltpu.sync_copy(dst2d.at[pl.ds(base, rows_per_tile)], B)

    def hix(i, _):
        j = i // 8
        k = (i % 8) * 16
        A[j, pl.ds(k, 16)] = A[j, pl.ds(k, 16)] * NP + B[j, pl.ds(k, 16)]
        return _
    lax.fori_loop(0, rows_per_tile * 8, hix, 0)

    def hstep(j, _):
        pltpu.async_copy(obuf, acch.at[A.at[j]], sem, add=True)
        return _
    lax.fori_loop(0, rows_per_tile, hstep, 0)

    def hstepw(j, _):
        pltpu.make_async_copy(obuf, acch.at[A.at[0]], sem).wait()
        return _
    lax.fori_loop(0, rows_per_tile, hstepw, 0)
    plsc.subcore_barrier()

    pltpu.sync_copy(acch.at[pl.ds(s * 2560, 2560)],
                    hist.at[c].at[pl.ds(s * 2560, 2560)])


_sc_hist = functools.partial(
    pl.kernel,
    out_type=jax.ShapeDtypeStruct((NC, R * NP, 16), jnp.float32),
    mesh=_mesh,
    scratch_types=[
        pltpu.VMEM((40, 128), jnp.int32),
        pltpu.VMEM((40, 128), jnp.int32),
        pltpu.VMEM((64, 16), jnp.float32),
        pltpu.VMEM((128, 16), jnp.float32),
        pltpu.VMEM_SHARED((R * NP, 16), jnp.float32),
        pltpu.SemaphoreType.DMA,
    ],
    compiler_params=_sc_params,
)(_sc_hist_body)


def _sc_ppi_body(et2d, src2d, dst2d, table32, out, A, B, C, rows32, rows32b,
                 zbuf32, acc32, sem, sem1, sem2, sem3):
    c = lax.axis_index("c")
    s = lax.axis_index("s")
    rows_per_tile = EROWS // NS
    base = s * rows_per_tile

    _zero_vmem(zbuf32, 64, 32)
    pltpu.sync_copy(et2d.at[pl.ds(base, rows_per_tile)], A)
    pltpu.sync_copy(src2d.at[pl.ds(base, rows_per_tile)], C)
    pltpu.sync_copy(dst2d.at[pl.ds(base, rows_per_tile)], B)

    def six(i, _):
        j = i // 8
        k = (i % 8) * 16
        B[j, pl.ds(k, 16)] = A[j, pl.ds(k, 16)] * NP + B[j, pl.ds(k, 16)]
        return _
    lax.fori_loop(0, rows_per_tile * 8, six, 0)

    def gix(i, _):
        j = i // 8
        k = (i % 8) * 16
        C[j, pl.ds(k, 16)] = (A[j, pl.ds(k, 16)] * N + C[j, pl.ds(k, 16)]) * 4
        return _
    lax.fori_loop(0, rows_per_tile * 8, gix, 0)

    for q in (0, 1):
        chunk = 2 * c + q

        def z1(j, _):
            pltpu.async_copy(zbuf32, acc32.at[pl.ds(s * 2560 + j * 64, 64)],
                             sem)
            return _
        lax.fori_loop(0, 40, z1, 0)

        def z1w(j, _):
            pltpu.make_async_copy(zbuf32, acc32.at[pl.ds(s * 2560, 64)],
                                  sem).wait()
            return _
        lax.fori_loop(0, 40, z1w, 0)

        def cix(i, _):
            j = i // 8
            k = (i % 8) * 16
            A[j, pl.ds(k, 16)] = C[j, pl.ds(k, 16)] + chunk
            return _
        lax.fori_loop(0, rows_per_tile * 8, cix, 0)
        plsc.subcore_barrier()

        _pipelined_gather_scatter(table32, A, B, rows32, rows32b, acc32,
                                  sem, sem1, sem2, sem3, rows_per_tile)
        plsc.subcore_barrier()

        pltpu.sync_copy(acc32.at[pl.ds(s * 2560, 2560)],
                        out.at[chunk].at[pl.ds(s * 2560, 2560)])
        plsc.subcore_barrier()


_sc_ppi = functools.partial(
    pl.kernel,
    out_type=jax.ShapeDtypeStruct((4, R * NP, 32), jnp.float32),
    mesh=_mesh,
    scratch_types=[
        pltpu.VMEM((80, 128), jnp.int32),
        pltpu.VMEM((80, 128), jnp.int32),
        pltpu.VMEM((80, 128), jnp.int32),
        pltpu.VMEM((128, 32), jnp.float32),
        pltpu.VMEM((128, 32), jnp.float32),
        pltpu.VMEM((64, 32), jnp.float32),
        pltpu.VMEM_SHARED((R * NP, 32), jnp.float32),
        pltpu.SemaphoreType.DMA,
        pltpu.SemaphoreType.DMA,
        pltpu.SemaphoreType.DMA,
        pltpu.SemaphoreType.DMA,
    ],
    compiler_params=_sc_params,
)(_sc_ppi_body)


def _relmm_body(x_ref, w_ref, o_ref):
    o_ref[...] = jnp.dot(x_ref[...], w_ref[0],
                         preferred_element_type=jnp.float32)[None]


def _relmm(x, W):
    return pl.pallas_call(
        _relmm_body,
        grid=(R, NB),
        in_specs=[
            pl.BlockSpec((BN, 128), lambda r, i: (i, 0)),
            pl.BlockSpec((1, 128, H), lambda r, i: (r, 0, 0)),
        ],
        out_specs=pl.BlockSpec((1, BN, H), lambda r, i: (r, i, 0)),
        out_shape=jax.ShapeDtypeStruct((R, N, H), jnp.float32),
        compiler_params=pltpu.CompilerParams(
            dimension_semantics=("parallel", "parallel")),
    )(x, W)


def _make_mgk(apply_relu):
    tdims = (((0,), (0,)), ((), ()))

    def body(x_ref, tiss_ref, mgx_ref, wmg_ref, wcross_ref, wpool_ref,
             relw_ref, msrc_ref, mdst_ref, met_ref,
             mgout_ref, mgt_ref, pool_acc, cnt_acc):
        i = pl.program_id(0)

        @pl.when(i == 0)
        def _():
            pool_acc[...] = jnp.zeros_like(pool_acc)
            cnt_acc[...] = jnp.zeros_like(cnt_acc)

        tb = tiss_ref[...]
        oh = (tb == lax.broadcasted_iota(jnp.int32, (BN, M), 1)).astype(
            jnp.float32)
        pool_acc[...] += lax.dot_general(
            oh, x_ref[...], tdims, preferred_element_type=jnp.float32)
        cnt_acc[...] += lax.dot_general(
            oh, jnp.ones((BN, 128), jnp.float32), tdims,
            preferred_element_type=jnp.float32)

        @pl.when(i == NB - 1)
        def _():
            mgx = mgx_ref[...]
            t = jnp.dot(mgx, wmg_ref[...], preferred_element_type=jnp.float32)
            msrc = msrc_ref[...]
            ohs = (msrc == lax.broadcasted_iota(jnp.int32, (EM, M), 1)).astype(
                jnp.float32)
            met = met_ref[...]
            ohe = (met == lax.broadcasted_iota(jnp.int32, (EM, R), 1)).astype(
                jnp.float32)
            mm = jnp.dot(ohs, t, preferred_element_type=jnp.float32) * jnp.dot(
                ohe, relw_ref[...], preferred_element_type=jnp.float32)
            mdst = mdst_ref[...]
            ohd = (mdst == lax.broadcasted_iota(jnp.int32, (EM, M), 1)).astype(
                jnp.float32)
            magg = lax.dot_general(ohd, mm, tdims,
                                   preferred_element_type=jnp.float32)
            mdeg = lax.dot_general(ohd, jnp.ones((EM, 128), jnp.float32),
                                   tdims, preferred_element_type=jnp.float32)
            pooled = pool_acc[...] / jnp.maximum(cnt_acc[...], 1.0)
            res = magg / jnp.maximum(mdeg, 1.0) + jnp.dot(
                pooled, wpool_ref[...], preferred_element_type=jnp.float32)
            if apply_relu:
                res = jnp.maximum(res, 0.0)
            mgout_ref[...] = res
            mgt_ref[...] = jnp.dot(mgx, wcross_ref[...],
                                   preferred_element_type=jnp.float32)

    return pl.pallas_call(
        body,
        grid=(NB,),
        in_specs=[
            pl.BlockSpec((BN, 128), lambda i: (i, 0)),
            pl.BlockSpec((BN, 1), lambda i: (i, 0)),
            pl.BlockSpec((M, 128), lambda i: (0, 0)),
            pl.BlockSpec((128, H), lambda i: (0, 0)),
            pl.BlockSpec((128, H), lambda i: (0, 0)),
            pl.BlockSpec((128, H), lambda i: (0, 0)),
            pl.BlockSpec((R, H), lambda i: (0, 0)),
            pl.BlockSpec((EM, 1), lambda i: (0, 0)),
            pl.BlockSpec((EM, 1), lambda i: (0, 0)),
            pl.BlockSpec((EM, 1), lambda i: (0, 0)),
        ],
        out_specs=[
            pl.BlockSpec((M, H), lambda i: (0, 0)),
            pl.BlockSpec((M, H), lambda i: (0, 0)),
        ],
        out_shape=[
            jax.ShapeDtypeStruct((M, H), jnp.float32),
            jax.ShapeDtypeStruct((M, H), jnp.float32),
        ],
        scratch_shapes=[
            pltpu.VMEM((M, 128), jnp.float32),
            pltpu.VMEM((M, 128), jnp.float32),
        ],
    )


def _pct_post_body(agg_ref, hist_ref, tiss_ref, mgt_ref, wq_ref, o_ref):
    h = hist_ref[...]
    deg = jnp.sum(h[..., 0], axis=(0, 1))
    agg = agg_ref[0] + agg_ref[1]
    t = tiss_ref[...]
    oh = (t == lax.broadcasted_iota(jnp.int32, (BN, M), 1)).astype(jnp.float32)
    p = agg / jnp.maximum(deg, 1.0)[:, None] + jnp.dot(
        oh, mgt_ref[...], preferred_element_type=jnp.float32)
    for r in range(R):
        o_ref[r] = jnp.dot(p, wq_ref[r], preferred_element_type=jnp.float32)


def _pct_post(agg, hist, tiss, mgt, Wq):
    return pl.pallas_call(
        _pct_post_body,
        grid=(NB,),
        in_specs=[
            pl.BlockSpec((NC, BN, 128), lambda i: (0, i, 0)),
            pl.BlockSpec((NC, R, BN, 16), lambda i: (0, 0, i, 0)),
            pl.BlockSpec((BN, 1), lambda i: (i, 0)),
            pl.BlockSpec((M, H), lambda i: (0, 0)),
            pl.BlockSpec((R, 128, H), lambda i: (0, 0, 0)),
        ],
        out_specs=pl.BlockSpec((R, BN, H), lambda i: (0, i, 0)),
        out_shape=jax.ShapeDtypeStruct((R, N, H), jnp.float32),
        compiler_params=pltpu.CompilerParams(
            dimension_semantics=("parallel",)),
    )(agg, hist, tiss, mgt, Wq)


def _ppi_post_body(agg_ref, hist_ref, wa_ref, ba_ref, qa_ref,
                   aggn_ref, wr_ref):
    i = pl.program_id(0)
    a = agg_ref[...]
    h = hist_ref[...]
    cnt = h[0, :, :, 0] + h[1, :, :, 0]
    full = jnp.concatenate([a[0], a[1], a[2], a[3]], axis=-1)
    aggn = full / jnp.maximum(cnt, 1.0)[:, :, None]
    aggn_ref[...] = aggn
    sco = jnp.tanh(
        jnp.dot(aggn.reshape(R * BN, H), wa_ref[...],
                preferred_element_type=jnp.float32) + ba_ref[...])
    pv = jnp.sum(sco * qa_ref[...], axis=-1).reshape(R, BN)
    s4 = jnp.sum(pv, axis=1)
    col0 = (lax.broadcasted_iota(jnp.int32, (R, 128), 1) == 0).astype(
        jnp.float32)
    contrib = jnp.concatenate([s4[:, None] * col0,
                               jnp.zeros((4, 128), jnp.float32)], axis=0)

    @pl.when(i == 0)
    def _():
        wr_ref[...] = jnp.zeros_like(wr_ref)

    wr_ref[...] += contrib


def _ppi_post(agg, hist, Wa, ba, qa):
    return pl.pallas_call(
        _ppi_post_body,
        grid=(NB,),
        in_specs=[
            pl.BlockSpec((4, R, BN, 32), lambda i: (0, 0, i, 0)),
            pl.BlockSpec((NC, R, BN, 16), lambda i: (0, 0, i, 0)),
            pl.BlockSpec((H, 8), lambda i: (0, 0)),
            pl.BlockSpec((1, 8), lambda i: (0, 0)),
            pl.BlockSpec((1, 8), lambda i: (0, 0)),
        ],
        out_specs=[
            pl.BlockSpec((R, BN, H), lambda i: (0, i, 0)),
            pl.BlockSpec((8, 128), lambda i: (0, 0)),
        ],
        out_shape=[
            jax.ShapeDtypeStruct((R, N, H), jnp.float32),
            jax.ShapeDtypeStruct((8, 128), jnp.float32),
        ],
    )(agg, hist, Wa, ba, qa)


def _beta_weighted(aggn_blk, wr_blk):
    w = wr_blk[:, 0:1] / float(N)
    rowmask = lax.broadcasted_iota(jnp.int32, (8, 1), 0) < R
    m = jnp.max(jnp.where(rowmask, w, -1e30))
    e = jnp.where(rowmask, jnp.exp(w - m), 0.0)
    beta = e / jnp.sum(e)
    return jnp.sum(aggn_blk * beta[0:R].reshape(R, 1, 1), axis=0)


def _combine_final(aggn, wr):
    def body(aggn_ref, wr_ref, o_ref):
        o_ref[...] = _beta_weighted(aggn_ref[...], wr_ref[...])

    return pl.pallas_call(
        body,
        grid=(NB,),
        in_specs=[
            pl.BlockSpec((R, BN, H), lambda i: (0, i, 0)),
            pl.BlockSpec((8, 128), lambda i: (0, 0)),
        ],
        out_specs=pl.BlockSpec((BN, H), lambda i: (i, 0)),
        out_shape=jax.ShapeDtypeStruct((N, H), jnp.float32),
        compiler_params=pltpu.CompilerParams(
            dimension_semantics=("parallel",)),
    )(aggn, wr)


def _combine_relmm(aggn, wr, Wp):
    def body(aggn_ref, wr_ref, wp_ref, xr_ref, p_ref):
        p = jnp.maximum(_beta_weighted(aggn_ref[...], wr_ref[...]), 0.0)
        p_ref[...] = p
        for r in range(R):
            xr_ref[r] = jnp.dot(p, wp_ref[r],
                                preferred_element_type=jnp.float32)

    return pl.pallas_call(
        body,
        grid=(NB,),
        in_specs=[
            pl.BlockSpec((R, BN, H), lambda i: (0, i, 0)),
            pl.BlockSpec((8, 128), lambda i: (0, 0)),
            pl.BlockSpec((R, 128, H), lambda i: (0, 0, 0)),
        ],
        out_specs=[
            pl.BlockSpec((R, BN, H), lambda i: (0, i, 0)),
            pl.BlockSpec((BN, H), lambda i: (i, 0)),
        ],
        out_shape=[
            jax.ShapeDtypeStruct((R, N, H), jnp.float32),
            jax.ShapeDtypeStruct((N, H), jnp.float32),
        ],
        compiler_params=pltpu.CompilerParams(
            dimension_semantics=("parallel",)),
    )(aggn, wr, Wp)


_mgk_relu = _make_mgk(True)
_mgk_final = _make_mgk(False)


def kernel(ppi_x, metagraph_x, ppi_edgetypes, mg_edgetypes, ppi_edge_index,
           mg_edge_index, tissue_neighbors, relw, Wp1, Wmg1, Wcross1, Wpool1,
           Wq1, Wa1, ba1, qa1, Wp2, Wmg2, Wcross2, Wpool2, Wq2, Wa2, ba2,
           qa2):
    et = ppi_edgetypes.astype(jnp.int32)
    src = ppi_edge_index[0].astype(jnp.int32)
    dst = ppi_edge_index[1].astype(jnp.int32)
    pad = EP - E
    et2d = jnp.concatenate([et, jnp.full((pad,), R - 1, jnp.int32)]
                           ).reshape(EROWS, 128)
    src2d = jnp.concatenate([src, jnp.zeros((pad,), jnp.int32)]
                            ).reshape(EROWS, 128)
    dst2d = jnp.concatenate([dst, jnp.full((pad,), N, jnp.int32)]
                            ).reshape(EROWS, 128)

    tiss = tissue_neighbors.astype(jnp.int32)
    tiss_col = tiss.reshape(N, 1)
    msrc = mg_edge_index[0].astype(jnp.int32).reshape(EM, 1)
    mdst_col = mg_edge_index[1].astype(jnp.int32).reshape(EM, 1)
    met = mg_edgetypes.astype(jnp.int32).reshape(EM, 1)
    ba1r = ba1.reshape(1, 8)
    qa1r = qa1.reshape(1, 8)
    ba2r = ba2.reshape(1, 8)
    qa2r = qa2.reshape(1, 8)

    hist4 = _sc_hist(et2d, dst2d).reshape(NC, R, NP, 16)

    xr1 = _relmm(ppi_x, Wp1)
    agg1 = _sc_pct(et2d, src2d, dst2d, xr1.reshape(R * N, H))
    mg1, mgt1 = _mgk_relu(ppi_x, tiss_col, metagraph_x, Wmg1, Wcross1,
                          Wpool1, relw, msrc, mdst_col, met)
    xr2_1 = _pct_post(agg1, hist4, tiss_col, mgt1, Wq1)
    agg2_1 = _sc_ppi(et2d, src2d, dst2d, xr2_1.reshape(R * N * 4, 32))
    aggn1, wr1 = _ppi_post(agg2_1.reshape(4, R, NP, 32), hist4, Wa1, ba1r,
                           qa1r)
    xr1_l2, p1relu = _combine_relmm(aggn1, wr1, Wp2)

    agg1_2 = _sc_pct(et2d, src2d, dst2d, xr1_l2.reshape(R * N, H))
    mg2, mgt2 = _mgk_final(p1relu, tiss_col, mg1, Wmg2, Wcross2, Wpool2,
                           relw, msrc, mdst_col, met)
    xr2_2 = _pct_post(agg1_2, hist4, tiss_col, mgt2, Wq2)
    agg2_2 = _sc_ppi(et2d, src2d, dst2d, xr2_2.reshape(R * N * 4, 32))
    aggn2, wr2 = _ppi_post(agg2_2.reshape(4, R, NP, 32), hist4, Wa2, ba2r,
                           qa2r)
    p2 = _combine_final(aggn2, wr2)
    return (p2, mg2)


# speedup vs baseline: 1.0799x; 1.0128x over previous
import functools

import jax
import jax.numpy as jnp
from jax import lax
from jax.experimental import pallas as pl
from jax.experimental.pallas import tpu as pltpu
from jax.experimental.pallas import tpu_sc as plsc

N = 10000
M = 200
E = 160000
EM = 2000
F = 128
H = 128
R = 4

NS = 16
NC = 2
NP = 10240
EP = 163840
EROWS = EP // 128
NB = 10
BN = 1000

_mesh = plsc.VectorSubcoreMesh(core_axis_name="c", subcore_axis_name="s")
_sc_params = pltpu.CompilerParams(use_tc_tiling_on_sc=False)


def _zero_vmem(buf, rows, width):
    per = width // 16

    def st(i, _):
        j = i // per
        k = (i % per) * 16
        buf[j, pl.ds(k, 16)] = jnp.zeros((16,), jnp.float32)
        return _

    lax.fori_loop(0, rows * per, st, 0)


def _pipelined_gather_scatter(table, A, B, rows0, rows1, acc,
                              semg0, semg1, sems0, sems1, K):
    pltpu.async_copy(table.at[A.at[0]], rows0, semg0)
    pltpu.async_copy(table.at[A.at[1]], rows1, semg1)

    def outer(g, carry):
        i0 = 2 * g
        pltpu.make_async_copy(table.at[A.at[i0]], rows0, semg0).wait()
        pltpu.async_copy(rows0, acc.at[B.at[i0]], sems0, add=True)
        pltpu.make_async_copy(table.at[A.at[i0 + 1]], rows1, semg1).wait()
        pltpu.async_copy(rows1, acc.at[B.at[i0 + 1]], sems1, add=True)
        pltpu.make_async_copy(rows0, acc.at[B.at[i0]], sems0).wait()

        @pl.when(i0 + 2 < K)
        def _f0():
            pltpu.async_copy(table.at[A.at[i0 + 2]], rows0, semg0)

        pltpu.make_async_copy(rows1, acc.at[B.at[i0 + 1]], sems1).wait()

        @pl.when(i0 + 3 < K)
        def _f1():
            pltpu.async_copy(table.at[A.at[i0 + 3]], rows1, semg1)

        return carry

    lax.fori_loop(0, K // 2, outer, 0)


def _sc_pct_body(et2d, src2d, dst2d, table, out, A, B, rows, rows1, zbuf,
                 acc, sem, sem1, sem2, sem3):
    c = lax.axis_index("c")
    s = lax.axis_index("s")
    wid = c * NS + s
    rows_per_tile = EROWS // (NC * NS)
    base = wid * rows_per_tile

    _zero_vmem(zbuf, 16, 128)

    def z1(j, _):
        pltpu.async_copy(zbuf, acc.at[pl.ds(s * 640 + j * 16, 16)], sem)
        return _
    lax.fori_loop(0, 40, z1, 0)

    def z1w(j, _):
        pltpu.make_async_copy(zbuf, acc.at[pl.ds(s * 640, 16)], sem).wait()
        return _
    lax.fori_loop(0, 40, z1w, 0)
    plsc.subcore_barrier()

    pltpu.sync_copy(et2d.at[pl.ds(base, rows_per_tile)], A)
    pltpu.sync_copy(src2d.at[pl.ds(base, rows_per_tile)], B)

    def gix(i, _):
        j = i // 8
        k = (i % 8) * 16
        A[j, pl.ds(k, 16)] = A[j, pl.ds(k, 16)] * N + B[j, pl.ds(k, 16)]
        return _
    lax.fori_loop(0, rows_per_tile * 8, gix, 0)
    pltpu.sync_copy(dst2d.at[pl.ds(base, rows_per_tile)], B)

    _pipelined_gather_scatter(table, A, B, rows, rows1, acc, sem, sem1,
                              sem2, sem3, rows_per_tile)
    plsc.subcore_barrier()

    pltpu.sync_copy(acc.at[pl.ds(s * 640, 640)],
                    out.at[c].at[pl.ds(s * 640, 640)])


_sc_pct = functools.partial(
    pl.kernel,
    out_type=jax.ShapeDtypeStruct((NC, NP, 128), jnp.float32),
    mesh=_mesh,
    scratch_types=[
        pltpu.VMEM((40, 128), jnp.int32),
        pltpu.VMEM((40, 128), jnp.int32),
        pltpu.VMEM((128, 128), jnp.float32),
        pltpu.VMEM((128, 128), jnp.float32),
        pltpu.VMEM((16, 128), jnp.float32),
        pltpu.VMEM_SHARED((NP, 128), jnp.float32),
        pltpu.SemaphoreType.DMA,
        pltpu.SemaphoreType.DMA,
        pltpu.SemaphoreType.DMA,
        pltpu.SemaphoreType.DMA,
    ],
    compiler_params=_sc_params,
)(_sc_pct_body)


def _sc_hist_body(et2d, dst2d, hist, A, B, zbuf, obuf, acch, sem):
    c = lax.axis_index("c")
    s = lax.axis_index("s")
    wid = c * NS + s
    rows_per_tile = EROWS // (NC * NS)
    base = wid * rows_per_tile

    _zero_vmem(zbuf, 64, 16)

    def z2(j, _):
        pltpu.async_copy(zbuf, acch.at[pl.ds(s * 2560 + j * 64, 64)], sem)
        return _
    lax.fori_loop(0, 40, z2, 0)

    def o1(j, _):
        obuf[j, pl.ds(0, 16)] = jnp.ones((16,), jnp.float32)
        return _
    lax.fori_loop(0, 128, o1, 0)

    def z2w(j, _):
        pltpu.make_async_copy(zbuf, acch.at[pl.ds(s * 2560, 64)], sem).wait()
        return _
    lax.fori_loop(0, 40, z2w, 0)
    plsc.subcore_barrier()

    pltpu.sync_copy(et2d.at[pl.ds(base, rows_per_tile)], A)
    pltpu.sync_copy(dst2d.at[pl.ds(base, rows_per_tile)], B)

    def hix(i, _):
        j = i // 8
        k = (i % 8) * 16
        A[j, pl.ds(k, 16)] = A[j, pl.ds(k, 16)] * NP + B[j, pl.ds(k, 16)]
        return _
    lax.fori_loop(0, rows_per_tile * 8, hix, 0)

    def hstep(j, _):
        pltpu.async_copy(obuf, acch.at[A.at[j]], sem, add=True)
        return _
    lax.fori_loop(0, rows_per_tile, hstep, 0)

    def hstepw(j, _):
        pltpu.make_async_copy(obuf, acch.at[A.at[0]], sem).wait()
        return _
    lax.fori_loop(0, rows_per_tile, hstepw, 0)
    plsc.subcore_barrier()

    pltpu.sync_copy(acch.at[pl.ds(s * 2560, 2560)],
                    hist.at[c].at[pl.ds(s * 2560, 2560)])


_sc_hist = functools.partial(
    pl.kernel,
    out_type=jax.ShapeDtypeStruct((NC, R * NP, 16), jnp.float32),
    mesh=_mesh,
    scratch_types=[
        pltpu.VMEM((40, 128), jnp.int32),
        pltpu.VMEM((40, 128), jnp.int32),
        pltpu.VMEM((64, 16), jnp.float32),
        pltpu.VMEM((128, 16), jnp.float32),
        pltpu.VMEM_SHARED((R * NP, 16), jnp.float32),
        pltpu.SemaphoreType.DMA,
    ],
    compiler_params=_sc_params,
)(_sc_hist_body)


def _sc_ppi_body(et2d, src2d, dst2d, table32, out, A, B, C, rows32, rows32b,
                 zbuf32, acc32, sem, sem1, sem2, sem3):
    c = lax.axis_index("c")
    s = lax.axis_index("s")
    rows_per_tile = EROWS // NS
    base = s * rows_per_tile

    _zero_vmem(zbuf32, 64, 32)
    pltpu.sync_copy(et2d.at[pl.ds(base, rows_per_tile)], A)
    pltpu.sync_copy(src2d.at[pl.ds(base, rows_per_tile)], C)
    pltpu.sync_copy(dst2d.at[pl.ds(base, rows_per_tile)], B)

    def six(i, _):
        j = i // 8
        k = (i % 8) * 16
        B[j, pl.ds(k, 16)] = A[j, pl.ds(k, 16)] * NP + B[j, pl.ds(k, 16)]
        return _
    lax.fori_loop(0, rows_per_tile * 8, six, 0)

    def gix(i, _):
        j = i // 8
        k = (i % 8) * 16
        C[j, pl.ds(k, 16)] = (A[j, pl.ds(k, 16)] * N + C[j, pl.ds(k, 16)]) * 4
        return _
    lax.fori_loop(0, rows_per_tile * 8, gix, 0)

    for q in (0, 1):
        chunk = 2 * c + q

        def z1(j, _):
            pltpu.async_copy(zbuf32, acc32.at[pl.ds(s * 2560 + j * 64, 64)],
                             sem)
            return _
        lax.fori_loop(0, 40, z1, 0)

        def z1w(j, _):
            pltpu.make_async_copy(zbuf32, acc32.at[pl.ds(s * 2560, 64)],
                                  sem).wait()
            return _
        lax.fori_loop(0, 40, z1w, 0)

        def cix(i, _):
            j = i // 8
            k = (i % 8) * 16
            A[j, pl.ds(k, 16)] = C[j, pl.ds(k, 16)] + chunk
            return _
        lax.fori_loop(0, rows_per_tile * 8, cix, 0)
        plsc.subcore_barrier()

        _pipelined_gather_scatter(table32, A, B, rows32, rows32b, acc32,
                                  sem, sem1, sem2, sem3, rows_per_tile)
        plsc.subcore_barrier()

        pltpu.sync_copy(acc32.at[pl.ds(s * 2560, 2560)],
                        out.at[chunk].at[pl.ds(s * 2560, 2560)])
        plsc.subcore_barrier()


_sc_ppi = functools.partial(
    pl.kernel,
    out_type=jax.ShapeDtypeStruct((4, R * NP, 32), jnp.float32),
    mesh=_mesh,
    scratch_types=[
        pltpu.VMEM((80, 128), jnp.int32),
        pltpu.VMEM((80, 128), jnp.int32),
        pltpu.VMEM((80, 128), jnp.int32),
        pltpu.VMEM((128, 32), jnp.float32),
        pltpu.VMEM((128, 32), jnp.float32),
        pltpu.VMEM((64, 32), jnp.float32),
        pltpu.VMEM_SHARED((R * NP, 32), jnp.float32),
        pltpu.SemaphoreType.DMA,
        pltpu.SemaphoreType.DMA,
        pltpu.SemaphoreType.DMA,
        pltpu.SemaphoreType.DMA,
    ],
    compiler_params=_sc_params,
)(_sc_ppi_body)


def _relmm_body(x_ref, w_ref, o_ref):
    xb = x_ref[...]
    for r in range(R):
        o_ref[r] = jnp.dot(xb, w_ref[r], preferred_element_type=jnp.float32)


def _relmm(x, W):
    return pl.pallas_call(
        _relmm_body,
        grid=(NB,),
        in_specs=[
            pl.BlockSpec((BN, 128), lambda i: (i, 0)),
            pl.BlockSpec((R, 128, H), lambda i: (0, 0, 0)),
        ],
        out_specs=pl.BlockSpec((R, BN, H), lambda i: (0, i, 0)),
        out_shape=jax.ShapeDtypeStruct((R, N, H), jnp.float32),
        compiler_params=pltpu.CompilerParams(
            dimension_semantics=("parallel",)),
    )(x, W)


def _make_mgk(apply_relu):
    tdims = (((0,), (0,)), ((), ()))

    def body(x_ref, tiss_ref, mgx_ref, wmg_ref, wcross_ref, wpool_ref,
             relw_ref, msrc_ref, mdst_ref, met_ref,
             mgout_ref, mgt_ref, pool_acc, cnt_acc):
        i = pl.program_id(0)

        @pl.when(i == 0)
        def _():
            pool_acc[...] = jnp.zeros_like(pool_acc)
            cnt_acc[...] = jnp.zeros_like(cnt_acc)

        tb = tiss_ref[...]
        oh = (tb == lax.broadcasted_iota(jnp.int32, (BN, M), 1)).astype(
            jnp.float32)
        pool_acc[...] += lax.dot_general(
            oh, x_ref[...], tdims, preferred_element_type=jnp.float32)
        cnt_acc[...] += lax.dot_general(
            oh, jnp.ones((BN, 128), jnp.float32), tdims,
            preferred_element_type=jnp.float32)

        @pl.when(i == NB - 1)
        def _():
            mgx = mgx_ref[...]
            t = jnp.dot(mgx, wmg_ref[...], preferred_element_type=jnp.float32)
            msrc = msrc_ref[...]
            ohs = (msrc == lax.broadcasted_iota(jnp.int32, (EM, M), 1)).astype(
                jnp.float32)
            met = met_ref[...]
            ohe = (met == lax.broadcasted_iota(jnp.int32, (EM, R), 1)).astype(
                jnp.float32)
            mm = jnp.dot(ohs, t, preferred_element_type=jnp.float32) * jnp.dot(
                ohe, relw_ref[...], preferred_element_type=jnp.float32)
            mdst = mdst_ref[...]
            ohd = (mdst == lax.broadcasted_iota(jnp.int32, (EM, M), 1)).astype(
                jnp.float32)
            magg = lax.dot_general(ohd, mm, tdims,
                                   preferred_element_type=jnp.float32)
            mdeg = lax.dot_general(ohd, jnp.ones((EM, 128), jnp.float32),
                                   tdims, preferred_element_type=jnp.float32)
            pooled = pool_acc[...] / jnp.maximum(cnt_acc[...], 1.0)
            res = magg / jnp.maximum(mdeg, 1.0) + jnp.dot(
                pooled, wpool_ref[...], preferred_element_type=jnp.float32)
            if apply_relu:
                res = jnp.maximum(res, 0.0)
            mgout_ref[...] = res
            mgt_ref[...] = jnp.dot(mgx, wcross_ref[...],
                                   preferred_element_type=jnp.float32)

    return pl.pallas_call(
        body,
        grid=(NB,),
        in_specs=[
            pl.BlockSpec((BN, 128), lambda i: (i, 0)),
            pl.BlockSpec((BN, 1), lambda i: (i, 0)),
            pl.BlockSpec((M, 128), lambda i: (0, 0)),
            pl.BlockSpec((128, H), lambda i: (0, 0)),
            pl.BlockSpec((128, H), lambda i: (0, 0)),
            pl.BlockSpec((128, H), lambda i: (0, 0)),
            pl.BlockSpec((R, H), lambda i: (0, 0)),
            pl.BlockSpec((EM, 1), lambda i: (0, 0)),
            pl.BlockSpec((EM, 1), lambda i: (0, 0)),
            pl.BlockSpec((EM, 1), lambda i: (0, 0)),
        ],
        out_specs=[
            pl.BlockSpec((M, H), lambda i: (0, 0)),
            pl.BlockSpec((M, H), lambda i: (0, 0)),
        ],
        out_shape=[
            jax.ShapeDtypeStruct((M, H), jnp.float32),
            jax.ShapeDtypeStruct((M, H), jnp.float32),
        ],
        scratch_shapes=[
            pltpu.VMEM((M, 128), jnp.float32),
            pltpu.VMEM((M, 128), jnp.float32),
        ],
    )


def _pct_post_body(agg_ref, hist_ref, tiss_ref, mgt_ref, wq_ref, o_ref):
    h = hist_ref[...]
    deg = jnp.sum(h[..., 0], axis=(0, 1))
    agg = agg_ref[0] + agg_ref[1]
    t = tiss_ref[...]
    oh = (t == lax.broadcasted_iota(jnp.int32, (BN, M), 1)).astype(jnp.float32)
    p = agg / jnp.maximum(deg, 1.0)[:, None] + jnp.dot(
        oh, mgt_ref[...], preferred_element_type=jnp.float32)
    for r in range(R):
        o_ref[r] = jnp.dot(p, wq_ref[r], preferred_element_type=jnp.float32)


def _pct_post(agg, hist, tiss, mgt, Wq):
    return pl.pallas_call(
        _pct_post_body,
        grid=(NB,),
        in_specs=[
            pl.BlockSpec((NC, BN, 128), lambda i: (0, i, 0)),
            pl.BlockSpec((NC, R, BN, 16), lambda i: (0, 0, i, 0)),
            pl.BlockSpec((BN, 1), lambda i: (i, 0)),
            pl.BlockSpec((M, H), lambda i: (0, 0)),
            pl.BlockSpec((R, 128, H), lambda i: (0, 0, 0)),
        ],
        out_specs=pl.BlockSpec((R, BN, H), lambda i: (0, i, 0)),
        out_shape=jax.ShapeDtypeStruct((R, N, H), jnp.float32),
        compiler_params=pltpu.CompilerParams(
            dimension_semantics=("parallel",)),
    )(agg, hist, tiss, mgt, Wq)


def _ppi_post_body(agg_ref, hist_ref, wa_ref, ba_ref, qa_ref,
                   aggn_ref, wr_ref):
    i = pl.program_id(0)
    a = agg_ref[...]
    h = hist_ref[...]
    cnt = h[0, :, :, 0] + h[1, :, :, 0]
    full = jnp.concatenate([a[0], a[1], a[2], a[3]], axis=-1)
    aggn = full / jnp.maximum(cnt, 1.0)[:, :, None]
    aggn_ref[...] = aggn
    sco = jnp.tanh(
        jnp.dot(aggn.reshape(R * BN, H), wa_ref[...],
                preferred_element_type=jnp.float32) + ba_ref[...])
    pv = jnp.sum(sco * qa_ref[...], axis=-1).reshape(R, BN)
    s4 = jnp.sum(pv, axis=1)
    col0 = (lax.broadcasted_iota(jnp.int32, (R, 128), 1) == 0).astype(
        jnp.float32)
    contrib = jnp.concatenate([s4[:, None] * col0,
                               jnp.zeros((4, 128), jnp.float32)], axis=0)

    @pl.when(i == 0)
    def _():
        wr_ref[...] = jnp.zeros_like(wr_ref)

    wr_ref[...] += contrib


def _ppi_post(agg, hist, Wa, ba, qa):
    return pl.pallas_call(
        _ppi_post_body,
        grid=(NB,),
        in_specs=[
            pl.BlockSpec((4, R, BN, 32), lambda i: (0, 0, i, 0)),
            pl.BlockSpec((NC, R, BN, 16), lambda i: (0, 0, i, 0)),
            pl.BlockSpec((H, 8), lambda i: (0, 0)),
            pl.BlockSpec((1, 8), lambda i: (0, 0)),
            pl.BlockSpec((1, 8), lambda i: (0, 0)),
        ],
        out_specs=[
            pl.BlockSpec((R, BN, H), lambda i: (0, i, 0)),
            pl.BlockSpec((8, 128), lambda i: (0, 0)),
        ],
        out_shape=[
            jax.ShapeDtypeStruct((R, N, H), jnp.float32),
            jax.ShapeDtypeStruct((8, 128), jnp.float32),
        ],
    )(agg, hist, Wa, ba, qa)


def _beta_weighted(aggn_blk, wr_blk):
    w = wr_blk[:, 0:1] / float(N)
    rowmask = lax.broadcasted_iota(jnp.int32, (8, 1), 0) < R
    m = jnp.max(jnp.where(rowmask, w, -1e30))
    e = jnp.where(rowmask, jnp.exp(w - m), 0.0)
    beta = e / jnp.sum(e)
    return jnp.sum(aggn_blk * beta[0:R].reshape(R, 1, 1), axis=0)


def _combine_final(aggn, wr):
    def body(aggn_ref, wr_ref, o_ref):
        o_ref[...] = _beta_weighted(aggn_ref[...], wr_ref[...])

    return pl.pallas_call(
        body,
        grid=(NB,),
        in_specs=[
            pl.BlockSpec((R, BN, H), lambda i: (0, i, 0)),
            pl.BlockSpec((8, 128), lambda i: (0, 0)),
        ],
        out_specs=pl.BlockSpec((BN, H), lambda i: (i, 0)),
        out_shape=jax.ShapeDtypeStruct((N, H), jnp.float32),
        compiler_params=pltpu.CompilerParams(
            dimension_semantics=("parallel",)),
    )(aggn, wr)


def _combine_relmm(aggn, wr, Wp):
    def body(aggn_ref, wr_ref, wp_ref, xr_ref, p_ref):
        p = jnp.maximum(_beta_weighted(aggn_ref[...], wr_ref[...]), 0.0)
        p_ref[...] = p
        for r in range(R):
            xr_ref[r] = jnp.dot(p, wp_ref[r],
                                preferred_element_type=jnp.float32)

    return pl.pallas_call(
        body,
        grid=(NB,),
        in_specs=[
            pl.BlockSpec((R, BN, H), lambda i: (0, i, 0)),
            pl.BlockSpec((8, 128), lambda i: (0, 0)),
            pl.BlockSpec((R, 128, H), lambda i: (0, 0, 0)),
        ],
        out_specs=[
            pl.BlockSpec((R, BN, H), lambda i: (0, i, 0)),
            pl.BlockSpec((BN, H), lambda i: (i, 0)),
        ],
        out_shape=[
            jax.ShapeDtypeStruct((R, N, H), jnp.float32),
            jax.ShapeDtypeStruct((N, H), jnp.float32),
        ],
        compiler_params=pltpu.CompilerParams(
            dimension_semantics=("parallel",)),
    )(aggn, wr, Wp)


_mgk_relu = _make_mgk(True)
_mgk_final = _make_mgk(False)


def kernel(ppi_x, metagraph_x, ppi_edgetypes, mg_edgetypes, ppi_edge_index,
           mg_edge_index, tissue_neighbors, relw, Wp1, Wmg1, Wcross1, Wpool1,
           Wq1, Wa1, ba1, qa1, Wp2, Wmg2, Wcross2, Wpool2, Wq2, Wa2, ba2,
           qa2):
    et = ppi_edgetypes.astype(jnp.int32)
    src = ppi_edge_index[0].astype(jnp.int32)
    dst = ppi_edge_index[1].astype(jnp.int32)
    pad = EP - E
    et2d = jnp.concatenate([et, jnp.full((pad,), R - 1, jnp.int32)]
                           ).reshape(EROWS, 128)
    src2d = jnp.concatenate([src, jnp.zeros((pad,), jnp.int32)]
                            ).reshape(EROWS, 128)
    dst2d = jnp.concatenate([dst, jnp.full((pad,), N, jnp.int32)]
                            ).reshape(EROWS, 128)

    tiss = tissue_neighbors.astype(jnp.int32)
    tiss_col = tiss.reshape(N, 1)
    msrc = mg_edge_index[0].astype(jnp.int32).reshape(EM, 1)
    mdst_col = mg_edge_index[1].astype(jnp.int32).reshape(EM, 1)
    met = mg_edgetypes.astype(jnp.int32).reshape(EM, 1)
    ba1r = ba1.reshape(1, 8)
    qa1r = qa1.reshape(1, 8)
    ba2r = ba2.reshape(1, 8)
    qa2r = qa2.reshape(1, 8)

    hist4 = _sc_hist(et2d, dst2d).reshape(NC, R, NP, 16)

    xr1 = _relmm(ppi_x, Wp1)
    agg1 = _sc_pct(et2d, src2d, dst2d, xr1.reshape(R * N, H))
    mg1, mgt1 = _mgk_relu(ppi_x, tiss_col, metagraph_x, Wmg1, Wcross1,
                          Wpool1, relw, msrc, mdst_col, met)
    xr2_1 = _pct_post(agg1, hist4, tiss_col, mgt1, Wq1)
    agg2_1 = _sc_ppi(et2d, src2d, dst2d, xr2_1.reshape(R * N * 4, 32))
    aggn1, wr1 = _ppi_post(agg2_1.reshape(4, R, NP, 32), hist4, Wa1, ba1r,
                           qa1r)
    xr1_l2, p1relu = _combine_relmm(aggn1, wr1, Wp2)

    agg1_2 = _sc_pct(et2d, src2d, dst2d, xr1_l2.reshape(R * N, H))
    mg2, mgt2 = _mgk_final(p1relu, tiss_col, mg1, Wmg2, Wcross2, Wpool2,
                           relw, msrc, mdst_col, met)
    xr2_2 = _pct_post(agg1_2, hist4, tiss_col, mgt2, Wq2)
    agg2_2 = _sc_ppi(et2d, src2d, dst2d, xr2_2.reshape(R * N * 4, 32))
    aggn2, wr2 = _ppi_post(agg2_2.reshape(4, R, NP, 32), hist4, Wa2, ba2r,
                           qa2r)
    p2 = _combine_final(aggn2, wr2)
    return (p2, mg2)
